# Initial kernel scaffold; baseline (speedup 1.0000x reference)
import jax
import jax.numpy as jnp
from jax.experimental import pallas as pl


def kernel(x, edge_index, edge_type, edge_pairs, rel_emb, W1, att_src1, att_dst1, b1, W2, att_src2, att_dst2, b2, Wm1, bm1, Wm2, bm2, Wm3, bm3):
    raise NotImplementedError("write your pallas kernel here")


# speedup vs baseline: 14.7845x; 14.7845x over previous
import functools

import jax
import jax.numpy as jnp
from jax import lax
from jax.experimental import pallas as pl
from jax.experimental.pallas import tpu as pltpu
from jax.experimental.pallas import tpu_sc as plsc

N = 10000
E = 160000
D = 128
HID = 128
OUT = 128
HEADS = 4
R = 16

NC = 2
NS = 16
NW = NC * NS
L = 16

C = 128
EPT = 5120
EPAD = EPT * NW
NCH_FULL = EPT // C
NCH_LAST = (E - EPT * (NW - 1)) // C

ROWS_A = 624
CNT_A = 10112
DEN_A = 5120

_mesh = plsc.VectorSubcoreMesh(core_axis_name="c", subcore_axis_name="s")


def _f32(shape):
    return jax.ShapeDtypeStruct(shape, jnp.float32)


def _wid_nch(c, s):
    wid = s * NC + c
    nch = jnp.where(wid == NW - 1, NCH_LAST, NCH_FULL)
    return wid, nch


def _fill_zeros_1d(ref, n):
    z = jnp.zeros((L,), jnp.float32)
    for i in range(n // L):
        ref[pl.ds(i * L, L)] = z


def _fill_zeros_rows(ref, rows):
    z = jnp.zeros((L,), jnp.float32)
    for r in range(rows):
        for j in range(128 // L):
            ref[r, pl.ds(j * L, L)] = z


def _zero_acc_rows(zrows_v, acc, s):
    base = s * ROWS_A
    off = 0
    for nrows in (64,) * 9 + (48,):
        pltpu.sync_copy(zrows_v.at[pl.ds(0, nrows)], acc.at[pl.ds(base + off, nrows)])
        off += nrows

    @pl.when(s == NS - 1)
    def _():
        pltpu.sync_copy(zrows_v.at[pl.ds(0, 16)],
                        acc.at[pl.ds(NS * ROWS_A, 16)])


def _zero_acc_flat(zflat_v, acc, s, words_a, words_last):
    base = s * words_a

    @pl.when(s < NS - 1)
    def _():
        for start in range(0, words_a, 1024):
            n = min(1024, words_a - start)
            pltpu.sync_copy(zflat_v.at[pl.ds(0, n)],
                            acc.at[pl.ds(base + start, n)])

    @pl.when(s == NS - 1)
    def _():
        b = (NS - 1) * words_a
        for start in range(0, words_last, 1024):
            n = min(1024, words_last - start)
            pltpu.sync_copy(zflat_v.at[pl.ds(0, n)],
                            acc.at[pl.ds(b + start, n)])


def _counts_body(src_hbm, et_hbm, cnt_out, sidx_v, etid_v, fidx_v, ones_v,
                 zflat_v, acc_cnt, sem):
    c = lax.axis_index("c")
    s = lax.axis_index("s")
    wid, nch = _wid_nch(c, s)

    _fill_zeros_1d(zflat_v, 1024)
    one = jnp.full((L,), 1.0, jnp.float32)
    for g in range(C // L):
        ones_v[pl.ds(g * L, L)] = one
    _zero_acc_flat(zflat_v, acc_cnt, s, CNT_A, N * R - (NS - 1) * CNT_A)
    plsc.subcore_barrier()

    def chunk(j, carry):
        base = wid * EPT + j * C
        pltpu.sync_copy(src_hbm.at[pl.ds(base, C)], sidx_v)
        pltpu.sync_copy(et_hbm.at[pl.ds(base, C)], etid_v)
        for g in range(C // L):
            sv = sidx_v[pl.ds(g * L, L)]
            ev = etid_v[pl.ds(g * L, L)]
            fidx_v[pl.ds(g * L, L)] = sv * R + ev
        pltpu.sync_copy(ones_v, acc_cnt.at[fidx_v], add=True)
        return carry

    lax.fori_loop(0, nch, chunk, 0)
    plsc.subcore_barrier()

    @pl.when(s < NS - 1)
    def _():
        pltpu.sync_copy(acc_cnt.at[pl.ds(s * CNT_A, CNT_A)],
                        cnt_out.at[pl.ds(c * (N * R) + s * CNT_A, CNT_A)])

    @pl.when(s == NS - 1)
    def _():
        tail = N * R - (NS - 1) * CNT_A
        pltpu.sync_copy(acc_cnt.at[pl.ds((NS - 1) * CNT_A, tail)],
                        cnt_out.at[pl.ds(c * (N * R) + (NS - 1) * CNT_A, tail)])


def _sc_counts(srcp, etp):
    return pl.kernel(
        _counts_body,
        out_type=_f32((NC * N * R,)),
        mesh=_mesh,
        scratch_types=[
            pltpu.VMEM((C,), jnp.int32),
            pltpu.VMEM((C,), jnp.int32),
            pltpu.VMEM((C,), jnp.int32),
            pltpu.VMEM((C,), jnp.float32),
            pltpu.VMEM((1024,), jnp.float32),
            pltpu.VMEM_SHARED((N * R,), jnp.float32),
            pltpu.SemaphoreType.DMA,
        ],
    )(srcp, etp)


def _gat_edges_body(nheads, src_hbm, dst_hbm, asf_hbm, adcf_hbm, hh_hbm,
                    num_out, den_out, sidx_v, didx_v, gidx_v, bidx_v, cidx_v,
                    aidx_v, as_v, ad_v, cs_v, rows_v, wbuf_v, zrows_v, zflat_v,
                    acc_num, acc_den, sem):
    c = lax.axis_index("c")
    s = lax.axis_index("s")
    wid, nch = _wid_nch(c, s)

    _fill_zeros_rows(zrows_v, 64)
    _fill_zeros_1d(zflat_v, 1024)
    _zero_acc_flat(zflat_v, acc_den, s, DEN_A, N * 8 - (NS - 1) * DEN_A)
    _zero_acc_rows(zrows_v, acc_num, s)
    plsc.subcore_barrier()

    def head_pass(h, carry):
        def chunk(j, carry2):
            base = wid * EPT + j * C
            pltpu.sync_copy(src_hbm.at[pl.ds(base, C)], sidx_v)
            pltpu.sync_copy(dst_hbm.at[pl.ds(base, C)], didx_v)
            for g in range(C // L):
                sl = pl.ds(g * L, L)
                sv = sidx_v[sl]
                dv = didx_v[sl]
                aidx_v[sl] = sv * 8 + h
                bidx_v[sl] = dv * 8 + h
                cidx_v[sl] = dv * 8 + (h + 4)
                gidx_v[sl] = sv + h * N
            a1 = pltpu.async_copy(asf_hbm.at[aidx_v], as_v, sem)
            a2 = pltpu.async_copy(adcf_hbm.at[bidx_v], ad_v, sem)
            a3 = pltpu.async_copy(adcf_hbm.at[cidx_v], cs_v, sem)
            a4 = pltpu.async_copy(hh_hbm.at[gidx_v], rows_v, sem)
            a1.wait()
            a2.wait()
            a3.wait()
            for g in range(C // L):
                sl = pl.ds(g * L, L)
                t = as_v[sl] + ad_v[sl]
                lk = jnp.maximum(t, 0.2 * t)
                wbuf_v[sl] = jnp.exp(lk - cs_v[sl])
            pltpu.sync_copy(wbuf_v, acc_den.at[bidx_v], add=True)
            a4.wait()

            def scale_grp(g, carry3):
                w16 = wbuf_v[pl.ds(g * L, L)]
                e0 = g * L
                for k in range(L):
                    wb = jnp.broadcast_to(w16[k], (L,))
                    for jj in range(128 // L):
                        sl = pl.ds(jj * L, L)
                        rows_v[e0 + k, sl] = rows_v[e0 + k, sl] * wb
                return carry3

            lax.fori_loop(0, C // L, scale_grp, 0)
            pltpu.sync_copy(rows_v, acc_num.at[didx_v], add=True)
            return carry2

        lax.fori_loop(0, nch, chunk, 0)
        plsc.subcore_barrier()
        base = s * ROWS_A

        @pl.when(s < NS - 1)
        def _():
            pltpu.sync_copy(acc_num.at[pl.ds(base, ROWS_A)],
                            num_out.at[c, h, pl.ds(base, ROWS_A)])

        @pl.when(s == NS - 1)
        def _():
            b = (NS - 1) * ROWS_A
            pltpu.sync_copy(acc_num.at[pl.ds(b, N - b)],
                            num_out.at[c, h, pl.ds(b, N - b)])

        plsc.subcore_barrier()
        _zero_acc_rows(zrows_v, acc_num, s)
        plsc.subcore_barrier()
        return carry

    lax.fori_loop(0, nheads, head_pass, 0)

    @pl.when(s < NS - 1)
    def _():
        pltpu.sync_copy(acc_den.at[pl.ds(s * DEN_A, DEN_A)],
                        den_out.at[pl.ds(c * (N * 8) + s * DEN_A, DEN_A)])

    @pl.when(s == NS - 1)
    def _():
        tail = N * 8 - (NS - 1) * DEN_A
        b = (NS - 1) * DEN_A
        pltpu.sync_copy(acc_den.at[pl.ds(b, tail)],
                        den_out.at[pl.ds(c * (N * 8) + b, tail)])


def _sc_gat_edges(srcp, dstp, ASf, ADCf, HHflat, nheads):
    num, den = pl.kernel(
        functools.partial(_gat_edges_body, nheads),
        out_type=[_f32((NC, nheads, N, 128)), _f32((NC * N * 8,))],
        mesh=_mesh,
        scratch_types=[
            pltpu.VMEM((C,), jnp.int32),
            pltpu.VMEM((C,), jnp.int32),
            pltpu.VMEM((C,), jnp.int32),
            pltpu.VMEM((C,), jnp.int32),
            pltpu.VMEM((C,), jnp.int32),
            pltpu.VMEM((C,), jnp.int32),
            pltpu.VMEM((C,), jnp.float32),
            pltpu.VMEM((C,), jnp.float32),
            pltpu.VMEM((C,), jnp.float32),
            pltpu.VMEM((C, 128), jnp.float32),
            pltpu.VMEM((C,), jnp.float32),
            pltpu.VMEM((64, 128), jnp.float32),
            pltpu.VMEM((1024,), jnp.float32),
            pltpu.VMEM_SHARED((N, 128), jnp.float32),
            pltpu.VMEM_SHARED((N * 8,), jnp.float32),
            pltpu.SemaphoreType.DMA,
        ],
    )(srcp, dstp, ASf, ADCf, HHflat)
    return num, den.reshape(NC, N, 8)


def _pairs_body(ep0_hbm, ep1_hbm, u_hbm, v_hbm, m_out, i0_v, i1_v, u_v, v_v, sem):
    c = lax.axis_index("c")
    s = lax.axis_index("s")
    wid, nch = _wid_nch(c, s)

    def chunk(j, carry):
        base = wid * EPT + j * C
        pltpu.sync_copy(ep0_hbm.at[pl.ds(base, C)], i0_v)
        pltpu.sync_copy(ep1_hbm.at[pl.ds(base, C)], i1_v)
        pltpu.async_copy(u_hbm.at[i0_v], u_v, sem).wait()
        pltpu.async_copy(v_hbm.at[i1_v], v_v, sem).wait()

        def addrelu(r, carry2):
            for jj in range(128 // L):
                sl = pl.ds(jj * L, L)
                u_v[r, sl] = jnp.maximum(u_v[r, sl] + v_v[r, sl], 0.0)
            return carry2

        lax.fori_loop(0, C, addrelu, 0)
        pltpu.sync_copy(u_v, m_out.at[pl.ds(base, C)])
        return carry

    lax.fori_loop(0, nch, chunk, 0)


def _sc_pairs(ep0p, ep1p, U, V):
    return pl.kernel(
        _pairs_body,
        out_type=_f32((EPAD, 128)),
        mesh=_mesh,
        scratch_types=[
            pltpu.VMEM((C,), jnp.int32),
            pltpu.VMEM((C,), jnp.int32),
            pltpu.VMEM((C, 128), jnp.float32),
            pltpu.VMEM((C, 128), jnp.float32),
            pltpu.SemaphoreType.DMA,
        ],
    )(ep0p, ep1p, U, V)


_NB = 1000


def _tc1_body(x_ref, cnt_ref, rel_ref, w1_ref, ats_ref, atd_ref,
              hh_ref, as_ref, adc_ref):
    cnt = cnt_ref[0] + cnt_ref[1]
    xm = x_ref[...] + cnt @ rel_ref[...]
    h = xm @ w1_ref[...]
    a_s = []
    a_d = []
    for k in range(HEADS):
        sl = slice(k * HID, (k + 1) * HID)
        hh_ref[k] = h[:, sl]
        a_s.append(jnp.sum(h[:, sl] * ats_ref[:, sl], axis=1, keepdims=True))
        a_d.append(jnp.sum(h[:, sl] * atd_ref[:, sl], axis=1, keepdims=True))
    a_s = jnp.concatenate(a_s, axis=1)
    a_d = jnp.concatenate(a_d, axis=1)
    t = a_s + a_d
    cst = jnp.maximum(t, 0.2 * t)
    as_ref[...] = jnp.concatenate([a_s, jnp.zeros_like(a_s)], axis=1)
    adc_ref[...] = jnp.concatenate([a_d, cst], axis=1)


def _tc1(x, cnt2, rel_emb, W1, ats1, atd1):
    return pl.pallas_call(
        _tc1_body,
        grid=(N // _NB,),
        in_specs=[
            pl.BlockSpec((_NB, D), lambda i: (i, 0)),
            pl.BlockSpec((NC, _NB, R), lambda i: (0, i, 0)),
            pl.BlockSpec((R, D), lambda i: (0, 0)),
            pl.BlockSpec((D, HEADS * HID), lambda i: (0, 0)),
            pl.BlockSpec((1, HEADS * HID), lambda i: (0, 0)),
            pl.BlockSpec((1, HEADS * HID), lambda i: (0, 0)),
        ],
        out_specs=[
            pl.BlockSpec((HEADS, _NB, HID), lambda i: (0, i, 0)),
            pl.BlockSpec((_NB, 8), lambda i: (i, 0)),
            pl.BlockSpec((_NB, 8), lambda i: (i, 0)),
        ],
        out_shape=[_f32((HEADS, N, HID)), _f32((N, 8)), _f32((N, 8))],
    )(x, cnt2, rel_emb, W1, ats1.reshape(1, -1), atd1.reshape(1, -1))


def _tc2_body(hh_ref, num_ref, den_ref, b1_ref, w2_ref, ats_ref, atd_ref,
              hh2_ref, as_ref, adc_ref):
    den = den_ref[0] + den_ref[1]
    gs = []
    for k in range(HEADS):
        nk = hh_ref[k] + num_ref[0, k] + num_ref[1, k]
        gk = nk / (1.0 + den[:, k:k + 1] + 1e-16) + b1_ref[:, k * HID:(k + 1) * HID]
        gk = jnp.where(gk > 0.0, gk, jnp.exp(jnp.minimum(gk, 0.0)) - 1.0)
        gs.append(gk)
    g = jnp.concatenate(gs, axis=1)
    h2 = g @ w2_ref[...]
    hh2_ref[...] = h2
    a_s = jnp.sum(h2 * ats_ref[...], axis=1, keepdims=True)
    a_d = jnp.sum(h2 * atd_ref[...], axis=1, keepdims=True)
    t = a_s + a_d
    cst = jnp.maximum(t, 0.2 * t)
    z4 = jnp.zeros_like(a_s)
    as_ref[...] = jnp.concatenate([a_s, z4, z4, z4, z4, z4, z4, z4], axis=1)
    adc_ref[...] = jnp.concatenate([a_d, z4, z4, z4, cst, z4, z4, z4], axis=1)


def _tc2(HH, NUM1, DEN1, b1, W2, ats2, atd2):
    return pl.pallas_call(
        _tc2_body,
        grid=(N // _NB,),
        in_specs=[
            pl.BlockSpec((HEADS, _NB, HID), lambda i: (0, i, 0)),
            pl.BlockSpec((NC, HEADS, _NB, HID), lambda i: (0, 0, i, 0)),
            pl.BlockSpec((NC, _NB, 8), lambda i: (0, i, 0)),
            pl.BlockSpec((1, HEADS * HID), lambda i: (0, 0)),
            pl.BlockSpec((HEADS * HID, OUT), lambda i: (0, 0)),
            pl.BlockSpec((1, OUT), lambda i: (0, 0)),
            pl.BlockSpec((1, OUT), lambda i: (0, 0)),
        ],
        out_specs=[
            pl.BlockSpec((_NB, OUT), lambda i: (i, 0)),
            pl.BlockSpec((_NB, 8), lambda i: (i, 0)),
            pl.BlockSpec((_NB, 8), lambda i: (i, 0)),
        ],
        out_shape=[_f32((N, OUT)), _f32((N, 8)), _f32((N, 8))],
    )(HH, NUM1, DEN1, b1.reshape(1, -1), W2, ats2.reshape(1, -1), atd2.reshape(1, -1))


def _tc3_body(hh2_ref, num_ref, den_ref, b2_ref, wa_ref, wb_ref, bm1_ref,
              u_ref, v_ref):
    den = den_ref[0] + den_ref[1]
    nk = hh2_ref[...] + num_ref[0] + num_ref[1]
    z = nk / (1.0 + den[:, 0:1] + 1e-16) + b2_ref[...]
    u_ref[...] = z @ wa_ref[...] + bm1_ref[...]
    v_ref[...] = z @ wb_ref[...]


def _tc3(HH2, NUM2, DEN2, b2, Wa, Wb, bm1):
    return pl.pallas_call(
        _tc3_body,
        grid=(N // _NB,),
        in_specs=[
            pl.BlockSpec((_NB, OUT), lambda i: (i, 0)),
            pl.BlockSpec((NC, _NB, OUT), lambda i: (0, i, 0)),
            pl.BlockSpec((NC, _NB, 8), lambda i: (0, i, 0)),
            pl.BlockSpec((1, OUT), lambda i: (0, 0)),
            pl.BlockSpec((OUT, OUT), lambda i: (0, 0)),
            pl.BlockSpec((OUT, OUT), lambda i: (0, 0)),
            pl.BlockSpec((1, OUT), lambda i: (0, 0)),
        ],
        out_specs=[
            pl.BlockSpec((_NB, OUT), lambda i: (i, 0)),
            pl.BlockSpec((_NB, OUT), lambda i: (i, 0)),
        ],
        out_shape=[_f32((N, OUT)), _f32((N, OUT))],
    )(HH2, NUM2, DEN2, b2.reshape(1, -1), Wa, Wb, bm1.reshape(1, -1))


_EB = 2000


def _tc4_body(m_ref, w2_ref, b2_ref, w3_ref, b3_ref, o_ref):
    m2 = jnp.maximum(m_ref[...] @ w2_ref[...] + b2_ref[...], 0.0)
    t = m2 @ w3_ref[...] + b3_ref[...]
    o_ref[...] = jax.nn.sigmoid(t)


def _tc4(M, Wm2, bm2, Wm3, bm3):
    w3 = jnp.pad(Wm3, ((0, 0), (0, 7)))
    b3 = jnp.pad(bm3.reshape(1, 1), ((0, 0), (0, 7)))
    out = pl.pallas_call(
        _tc4_body,
        grid=(E // _EB,),
        in_specs=[
            pl.BlockSpec((_EB, OUT), lambda i: (i, 0)),
            pl.BlockSpec((OUT, OUT // 2), lambda i: (0, 0)),
            pl.BlockSpec((1, OUT // 2), lambda i: (0, 0)),
            pl.BlockSpec((OUT // 2, 8), lambda i: (0, 0)),
            pl.BlockSpec((1, 8), lambda i: (0, 0)),
        ],
        out_specs=pl.BlockSpec((_EB, 8), lambda i: (i, 0)),
        out_shape=_f32((E, 8)),
    )(M, Wm2, bm2.reshape(1, -1), w3, b3)
    return out[:, :1]


def _pad_idx(a):
    return jnp.concatenate([a, jnp.zeros((EPAD - E,), a.dtype)])


def kernel(x, edge_index, edge_type, edge_pairs, rel_emb, W1, att_src1, att_dst1,
           b1, W2, att_src2, att_dst2, b2, Wm1, bm1, Wm2, bm2, Wm3, bm3):
    src = edge_index[0].astype(jnp.int32)
    dst = edge_index[1].astype(jnp.int32)
    et = edge_type.astype(jnp.int32)
    srcp, dstp, etp = _pad_idx(src), _pad_idx(dst), _pad_idx(et)

    cnt2 = _sc_counts(srcp, etp).reshape(NC, N, R)
    HH, AS1, ADC1 = _tc1(x, cnt2, rel_emb, W1, att_src1, att_dst1)
    NUM1, DEN1 = _sc_gat_edges(srcp, dstp, AS1.reshape(-1), ADC1.reshape(-1),
                               HH.reshape(HEADS * N, HID), HEADS)
    HH2, AS2, ADC2 = _tc2(HH, NUM1, DEN1, b1, W2, att_src2, att_dst2)
    NUM2, DEN2 = _sc_gat_edges(srcp, dstp, AS2.reshape(-1), ADC2.reshape(-1),
                               HH2, 1)
    U, V = _tc3(HH2, NUM2.reshape(NC, N, OUT), DEN2, b2, Wm1[:OUT], Wm1[OUT:], bm1)
    ep0 = _pad_idx(edge_pairs[0].astype(jnp.int32))
    ep1 = _pad_idx(edge_pairs[1].astype(jnp.int32))
    M = _sc_pairs(ep0, ep1, U, V)
    return _tc4(M[:E], Wm2, bm2, Wm3, bm3)


# speedup vs baseline: 17.7928x; 1.2035x over previous
import functools

import jax
import jax.numpy as jnp
from jax import lax
from jax.experimental import pallas as pl
from jax.experimental.pallas import tpu as pltpu
from jax.experimental.pallas import tpu_sc as plsc

N = 10000
E = 160000
D = 128
HID = 128
OUT = 128
HEADS = 4
R = 16

NC = 2
NS = 16
NW = NC * NS
L = 16

C = 128
EPT = 5120
EPAD = EPT * NW
NCH_FULL = EPT // C
NCH_LAST = (E - EPT * (NW - 1)) // C

ROWS_A = 624
CNT_A = 10112
DEN_A = 5120

_mesh = plsc.VectorSubcoreMesh(core_axis_name="c", subcore_axis_name="s")


def _f32(shape):
    return jax.ShapeDtypeStruct(shape, jnp.float32)


def _wid_nch(c, s):
    wid = s * NC + c
    nch = jnp.where(wid == NW - 1, NCH_LAST, NCH_FULL)
    return wid, nch


def _fill_zeros_1d(ref, n):
    z = jnp.zeros((L,), jnp.float32)
    for i in range(n // L):
        ref[pl.ds(i * L, L)] = z


def _fill_zeros_rows(ref, rows):
    z = jnp.zeros((L,), jnp.float32)
    for r in range(rows):
        for j in range(128 // L):
            ref[r, pl.ds(j * L, L)] = z


def _zero_acc_rows(zrows_v, acc, s):
    base = s * ROWS_A
    off = 0
    for nrows in (64,) * 9 + (48,):
        pltpu.sync_copy(zrows_v.at[pl.ds(0, nrows)], acc.at[pl.ds(base + off, nrows)])
        off += nrows

    @pl.when(s == NS - 1)
    def _():
        pltpu.sync_copy(zrows_v.at[pl.ds(0, 16)],
                        acc.at[pl.ds(NS * ROWS_A, 16)])


def _zero_acc_flat(zflat_v, acc, s, words_a, words_last):
    base = s * words_a

    @pl.when(s < NS - 1)
    def _():
        for start in range(0, words_a, 1024):
            n = min(1024, words_a - start)
            pltpu.sync_copy(zflat_v.at[pl.ds(0, n)],
                            acc.at[pl.ds(base + start, n)])

    @pl.when(s == NS - 1)
    def _():
        b = (NS - 1) * words_a
        for start in range(0, words_last, 1024):
            n = min(1024, words_last - start)
            pltpu.sync_copy(zflat_v.at[pl.ds(0, n)],
                            acc.at[pl.ds(b + start, n)])


def _counts_body(src_hbm, et_hbm, cnt_out, sidx_v, etid_v, fidx_v, ones_v,
                 zflat_v, acc_cnt, sem):
    c = lax.axis_index("c")
    s = lax.axis_index("s")
    wid, nch = _wid_nch(c, s)

    _fill_zeros_1d(zflat_v, 1024)
    one = jnp.full((L,), 1.0, jnp.float32)
    for g in range(C // L):
        ones_v[pl.ds(g * L, L)] = one
    _zero_acc_flat(zflat_v, acc_cnt, s, CNT_A, N * R - (NS - 1) * CNT_A)
    plsc.subcore_barrier()

    def chunk(j, carry):
        base = wid * EPT + j * C
        pltpu.sync_copy(src_hbm.at[pl.ds(base, C)], sidx_v)
        pltpu.sync_copy(et_hbm.at[pl.ds(base, C)], etid_v)
        for g in range(C // L):
            sv = sidx_v[pl.ds(g * L, L)]
            ev = etid_v[pl.ds(g * L, L)]
            fidx_v[pl.ds(g * L, L)] = sv * R + ev
        pltpu.sync_copy(ones_v, acc_cnt.at[fidx_v], add=True)
        return carry

    lax.fori_loop(0, nch, chunk, 0)
    plsc.subcore_barrier()

    @pl.when(s < NS - 1)
    def _():
        pltpu.sync_copy(acc_cnt.at[pl.ds(s * CNT_A, CNT_A)],
                        cnt_out.at[pl.ds(c * (N * R) + s * CNT_A, CNT_A)])

    @pl.when(s == NS - 1)
    def _():
        tail = N * R - (NS - 1) * CNT_A
        pltpu.sync_copy(acc_cnt.at[pl.ds((NS - 1) * CNT_A, tail)],
                        cnt_out.at[pl.ds(c * (N * R) + (NS - 1) * CNT_A, tail)])


def _sc_counts(srcp, etp):
    return pl.kernel(
        _counts_body,
        out_type=_f32((NC * N * R,)),
        mesh=_mesh,
        scratch_types=[
            pltpu.VMEM((C,), jnp.int32),
            pltpu.VMEM((C,), jnp.int32),
            pltpu.VMEM((C,), jnp.int32),
            pltpu.VMEM((C,), jnp.float32),
            pltpu.VMEM((1024,), jnp.float32),
            pltpu.VMEM_SHARED((N * R,), jnp.float32),
            pltpu.SemaphoreType.DMA,
        ],
    )(srcp, etp)


def _gat_edges_body(nheads, src_hbm, dst_hbm, asf_hbm, adcf_hbm, hh_hbm,
                    num_out, den_out, sidx_v, didx_v, gidx_v, bidx_v, cidx_v,
                    aidx_v, as_v, ad_v, cs_v, rows_v, wbuf_v, zrows_v, zflat_v,
                    acc_num, acc_den, sem):
    c = lax.axis_index("c")
    s = lax.axis_index("s")
    wid, nch = _wid_nch(c, s)

    _fill_zeros_rows(zrows_v, 64)
    _fill_zeros_1d(zflat_v, 1024)
    _zero_acc_flat(zflat_v, acc_den, s, DEN_A, N * 8 - (NS - 1) * DEN_A)
    _zero_acc_rows(zrows_v, acc_num, s)
    plsc.subcore_barrier()

    def head_pass(h, carry):
        def chunk(j, carry2):
            base = wid * EPT + j * C
            pltpu.sync_copy(src_hbm.at[pl.ds(base, C)], sidx_v)
            pltpu.sync_copy(dst_hbm.at[pl.ds(base, C)], didx_v)
            for g in range(C // L):
                sl = pl.ds(g * L, L)
                sv = sidx_v[sl]
                dv = didx_v[sl]
                aidx_v[sl] = sv * 8 + h
                bidx_v[sl] = dv * 8 + h
                cidx_v[sl] = dv * 8 + (h + 4)
                gidx_v[sl] = sv + h * N
            a1 = pltpu.async_copy(asf_hbm.at[aidx_v], as_v, sem)
            a2 = pltpu.async_copy(adcf_hbm.at[bidx_v], ad_v, sem)
            a3 = pltpu.async_copy(adcf_hbm.at[cidx_v], cs_v, sem)
            a4 = pltpu.async_copy(hh_hbm.at[gidx_v], rows_v, sem)
            a1.wait()
            a2.wait()
            a3.wait()
            for g in range(C // L):
                sl = pl.ds(g * L, L)
                t = as_v[sl] + ad_v[sl]
                lk = jnp.maximum(t, 0.2 * t)
                wbuf_v[sl] = jnp.exp(lk - cs_v[sl])
            pltpu.sync_copy(wbuf_v, acc_den.at[bidx_v], add=True)
            a4.wait()

            def scale_grp(g, carry3):
                w16 = wbuf_v[pl.ds(g * L, L)]
                e0 = g * L
                for k in range(L):
                    wb = jnp.broadcast_to(w16[k], (L,))
                    for jj in range(128 // L):
                        sl = pl.ds(jj * L, L)
                        rows_v[e0 + k, sl] = rows_v[e0 + k, sl] * wb
                return carry3

            lax.fori_loop(0, C // L, scale_grp, 0)
            pltpu.sync_copy(rows_v, acc_num.at[didx_v], add=True)
            return carry2

        lax.fori_loop(0, nch, chunk, 0)
        plsc.subcore_barrier()
        base = s * ROWS_A

        @pl.when(s < NS - 1)
        def _():
            pltpu.sync_copy(acc_num.at[pl.ds(base, ROWS_A)],
                            num_out.at[c, h, pl.ds(base, ROWS_A)])

        @pl.when(s == NS - 1)
        def _():
            b = (NS - 1) * ROWS_A
            pltpu.sync_copy(acc_num.at[pl.ds(b, N - b)],
                            num_out.at[c, h, pl.ds(b, N - b)])

        plsc.subcore_barrier()
        _zero_acc_rows(zrows_v, acc_num, s)
        plsc.subcore_barrier()
        return carry

    lax.fori_loop(0, nheads, head_pass, 0)

    @pl.when(s < NS - 1)
    def _():
        pltpu.sync_copy(acc_den.at[pl.ds(s * DEN_A, DEN_A)],
                        den_out.at[pl.ds(c * (N * 8) + s * DEN_A, DEN_A)])

    @pl.when(s == NS - 1)
    def _():
        tail = N * 8 - (NS - 1) * DEN_A
        b = (NS - 1) * DEN_A
        pltpu.sync_copy(acc_den.at[pl.ds(b, tail)],
                        den_out.at[pl.ds(c * (N * 8) + b, tail)])


def _gat_edges_body2(nheads, src_hbm, dst_hbm, asf_hbm, adcf_hbm, hh_hbm,
                     num_out, den_out, bufs0, bufs1, wbuf_v, zrows_v, zflat_v,
                     acc_num, acc_den, sem0, sem1):
    c = lax.axis_index("c")
    s = lax.axis_index("s")
    wid, nch = _wid_nch(c, s)

    _fill_zeros_rows(zrows_v, 64)
    _fill_zeros_1d(zflat_v, 1024)
    _zero_acc_flat(zflat_v, acc_den, s, DEN_A, N * 8 - (NS - 1) * DEN_A)
    _zero_acc_rows(zrows_v, acc_num, s)
    plsc.subcore_barrier()

    def issue(j, h, bufs, sem):
        (sidx_v, didx_v, gidx_v, bidx_v, cidx_v, aidx_v,
         as_v, ad_v, cs_v, rows_v) = bufs
        base = wid * EPT + j * C
        pltpu.sync_copy(src_hbm.at[pl.ds(base, C)], sidx_v)
        pltpu.sync_copy(dst_hbm.at[pl.ds(base, C)], didx_v)
        for g in range(C // L):
            sl = pl.ds(g * L, L)
            sv = sidx_v[sl]
            dv = didx_v[sl]
            aidx_v[sl] = sv * 8 + h
            bidx_v[sl] = dv * 8 + h
            cidx_v[sl] = dv * 8 + (h + 4)
            gidx_v[sl] = sv + h * N
        a1 = pltpu.async_copy(asf_hbm.at[aidx_v], as_v, sem)
        a2 = pltpu.async_copy(adcf_hbm.at[bidx_v], ad_v, sem)
        a3 = pltpu.async_copy(adcf_hbm.at[cidx_v], cs_v, sem)
        a4 = pltpu.async_copy(hh_hbm.at[gidx_v], rows_v, sem)
        return (a1, a2, a3, a4)

    def consume(hnds, bufs):
        (sidx_v, didx_v, gidx_v, bidx_v, cidx_v, aidx_v,
         as_v, ad_v, cs_v, rows_v) = bufs
        a1, a2, a3, a4 = hnds
        a1.wait()
        a2.wait()
        a3.wait()
        for g in range(C // L):
            sl = pl.ds(g * L, L)
            t = as_v[sl] + ad_v[sl]
            lk = jnp.maximum(t, 0.2 * t)
            wbuf_v[sl] = jnp.exp(lk - cs_v[sl])
        pltpu.sync_copy(wbuf_v, acc_den.at[bidx_v], add=True)
        a4.wait()

        def scale_grp(g, carry3):
            w16 = wbuf_v[pl.ds(g * L, L)]
            e0 = g * L
            for k in range(L):
                wb = jnp.broadcast_to(w16[k], (L,))
                for jj in range(128 // L):
                    sl = pl.ds(jj * L, L)
                    rows_v[e0 + k, sl] = rows_v[e0 + k, sl] * wb
            return carry3

        lax.fori_loop(0, C // L, scale_grp, 0)
        pltpu.sync_copy(rows_v, acc_num.at[didx_v], add=True)

    def head_pass(h, carry):
        def pair(t, carry2):
            h0 = issue(2 * t, h, bufs0, sem0)
            h1 = issue(2 * t + 1, h, bufs1, sem1)
            consume(h0, bufs0)
            consume(h1, bufs1)
            return carry2

        lax.fori_loop(0, nch // 2, pair, 0)
        plsc.subcore_barrier()
        base = s * ROWS_A

        @pl.when(s < NS - 1)
        def _():
            pltpu.sync_copy(acc_num.at[pl.ds(base, ROWS_A)],
                            num_out.at[c, h, pl.ds(base, ROWS_A)])

        @pl.when(s == NS - 1)
        def _():
            b = (NS - 1) * ROWS_A
            pltpu.sync_copy(acc_num.at[pl.ds(b, N - b)],
                            num_out.at[c, h, pl.ds(b, N - b)])

        plsc.subcore_barrier()
        _zero_acc_rows(zrows_v, acc_num, s)
        plsc.subcore_barrier()
        return carry

    lax.fori_loop(0, nheads, head_pass, 0)

    @pl.when(s < NS - 1)
    def _():
        pltpu.sync_copy(acc_den.at[pl.ds(s * DEN_A, DEN_A)],
                        den_out.at[pl.ds(c * (N * 8) + s * DEN_A, DEN_A)])

    @pl.when(s == NS - 1)
    def _():
        tail = N * 8 - (NS - 1) * DEN_A
        b = (NS - 1) * DEN_A
        pltpu.sync_copy(acc_den.at[pl.ds(b, tail)],
                        den_out.at[pl.ds(c * (N * 8) + b, tail)])


def _edge_buf_types():
    return [
        pltpu.VMEM((C,), jnp.int32),
        pltpu.VMEM((C,), jnp.int32),
        pltpu.VMEM((C,), jnp.int32),
        pltpu.VMEM((C,), jnp.int32),
        pltpu.VMEM((C,), jnp.int32),
        pltpu.VMEM((C,), jnp.int32),
        pltpu.VMEM((C,), jnp.float32),
        pltpu.VMEM((C,), jnp.float32),
        pltpu.VMEM((C,), jnp.float32),
        pltpu.VMEM((C, 128), jnp.float32),
    ]


def _gat_edges_wrap(nheads, src_hbm, dst_hbm, asf_hbm, adcf_hbm, hh_hbm,
                    num_out, den_out, *scratch):
    bufs0 = scratch[0:10]
    bufs1 = scratch[10:20]
    rest = scratch[20:]
    _gat_edges_body2(nheads, src_hbm, dst_hbm, asf_hbm, adcf_hbm, hh_hbm,
                     num_out, den_out, bufs0, bufs1, *rest)


def _sc_gat_edges(srcp, dstp, ASf, ADCf, HHflat, nheads):
    num, den = pl.kernel(
        functools.partial(_gat_edges_wrap, nheads),
        out_type=[_f32((NC, nheads, N, 128)), _f32((NC * N * 8,))],
        mesh=_mesh,
        scratch_types=_edge_buf_types() + _edge_buf_types() + [
            pltpu.VMEM((C,), jnp.float32),
            pltpu.VMEM((64, 128), jnp.float32),
            pltpu.VMEM((1024,), jnp.float32),
            pltpu.VMEM_SHARED((N, 128), jnp.float32),
            pltpu.VMEM_SHARED((N * 8,), jnp.float32),
            pltpu.SemaphoreType.DMA,
            pltpu.SemaphoreType.DMA,
        ],
    )(srcp, dstp, ASf, ADCf, HHflat)
    return num, den.reshape(NC, N, 8)


def _pairs_body(ep0_hbm, ep1_hbm, u_hbm, v_hbm, m_out, i0_v, i1_v, u_v, v_v, sem):
    c = lax.axis_index("c")
    s = lax.axis_index("s")
    wid, nch = _wid_nch(c, s)

    def chunk(j, carry):
        base = wid * EPT + j * C
        pltpu.sync_copy(ep0_hbm.at[pl.ds(base, C)], i0_v)
        pltpu.sync_copy(ep1_hbm.at[pl.ds(base, C)], i1_v)
        pltpu.async_copy(u_hbm.at[i0_v], u_v, sem).wait()
        pltpu.async_copy(v_hbm.at[i1_v], v_v, sem).wait()

        def addrelu(r, carry2):
            for jj in range(128 // L):
                sl = pl.ds(jj * L, L)
                u_v[r, sl] = jnp.maximum(u_v[r, sl] + v_v[r, sl], 0.0)
            return carry2

        lax.fori_loop(0, C, addrelu, 0)
        pltpu.sync_copy(u_v, m_out.at[pl.ds(base, C)])
        return carry

    lax.fori_loop(0, nch, chunk, 0)


def _sc_pairs(ep0p, ep1p, U, V):
    return pl.kernel(
        _pairs_body,
        out_type=_f32((EPAD, 128)),
        mesh=_mesh,
        scratch_types=[
            pltpu.VMEM((C,), jnp.int32),
            pltpu.VMEM((C,), jnp.int32),
            pltpu.VMEM((C, 128), jnp.float32),
            pltpu.VMEM((C, 128), jnp.float32),
            pltpu.SemaphoreType.DMA,
        ],
    )(ep0p, ep1p, U, V)


_NB = 1000


def _tc1_body(x_ref, cnt_ref, rel_ref, w1_ref, ats_ref, atd_ref,
              hh_ref, as_ref, adc_ref):
    cnt = cnt_ref[0] + cnt_ref[1]
    xm = x_ref[...] + cnt @ rel_ref[...]
    h = xm @ w1_ref[...]
    a_s = []
    a_d = []
    for k in range(HEADS):
        sl = slice(k * HID, (k + 1) * HID)
        hh_ref[k] = h[:, sl]
        a_s.append(jnp.sum(h[:, sl] * ats_ref[:, sl], axis=1, keepdims=True))
        a_d.append(jnp.sum(h[:, sl] * atd_ref[:, sl], axis=1, keepdims=True))
    a_s = jnp.concatenate(a_s, axis=1)
    a_d = jnp.concatenate(a_d, axis=1)
    t = a_s + a_d
    cst = jnp.maximum(t, 0.2 * t)
    as_ref[...] = jnp.concatenate([a_s, jnp.zeros_like(a_s)], axis=1)
    adc_ref[...] = jnp.concatenate([a_d, cst], axis=1)


def _tc1(x, cnt2, rel_emb, W1, ats1, atd1):
    return pl.pallas_call(
        _tc1_body,
        grid=(N // _NB,),
        in_specs=[
            pl.BlockSpec((_NB, D), lambda i: (i, 0)),
            pl.BlockSpec((NC, _NB, R), lambda i: (0, i, 0)),
            pl.BlockSpec((R, D), lambda i: (0, 0)),
            pl.BlockSpec((D, HEADS * HID), lambda i: (0, 0)),
            pl.BlockSpec((1, HEADS * HID), lambda i: (0, 0)),
            pl.BlockSpec((1, HEADS * HID), lambda i: (0, 0)),
        ],
        out_specs=[
            pl.BlockSpec((HEADS, _NB, HID), lambda i: (0, i, 0)),
            pl.BlockSpec((_NB, 8), lambda i: (i, 0)),
            pl.BlockSpec((_NB, 8), lambda i: (i, 0)),
        ],
        out_shape=[_f32((HEADS, N, HID)), _f32((N, 8)), _f32((N, 8))],
    )(x, cnt2, rel_emb, W1, ats1.reshape(1, -1), atd1.reshape(1, -1))


def _tc2_body(hh_ref, num_ref, den_ref, b1_ref, w2_ref, ats_ref, atd_ref,
              hh2_ref, as_ref, adc_ref):
    den = den_ref[0] + den_ref[1]
    gs = []
    for k in range(HEADS):
        nk = hh_ref[k] + num_ref[0, k] + num_ref[1, k]
        gk = nk / (1.0 + den[:, k:k + 1] + 1e-16) + b1_ref[:, k * HID:(k + 1) * HID]
        gk = jnp.where(gk > 0.0, gk, jnp.exp(jnp.minimum(gk, 0.0)) - 1.0)
        gs.append(gk)
    g = jnp.concatenate(gs, axis=1)
    h2 = g @ w2_ref[...]
    hh2_ref[...] = h2
    a_s = jnp.sum(h2 * ats_ref[...], axis=1, keepdims=True)
    a_d = jnp.sum(h2 * atd_ref[...], axis=1, keepdims=True)
    t = a_s + a_d
    cst = jnp.maximum(t, 0.2 * t)
    z4 = jnp.zeros_like(a_s)
    as_ref[...] = jnp.concatenate([a_s, z4, z4, z4, z4, z4, z4, z4], axis=1)
    adc_ref[...] = jnp.concatenate([a_d, z4, z4, z4, cst, z4, z4, z4], axis=1)


def _tc2(HH, NUM1, DEN1, b1, W2, ats2, atd2):
    return pl.pallas_call(
        _tc2_body,
        grid=(N // _NB,),
        in_specs=[
            pl.BlockSpec((HEADS, _NB, HID), lambda i: (0, i, 0)),
            pl.BlockSpec((NC, HEADS, _NB, HID), lambda i: (0, 0, i, 0)),
            pl.BlockSpec((NC, _NB, 8), lambda i: (0, i, 0)),
            pl.BlockSpec((1, HEADS * HID), lambda i: (0, 0)),
            pl.BlockSpec((HEADS * HID, OUT), lambda i: (0, 0)),
            pl.BlockSpec((1, OUT), lambda i: (0, 0)),
            pl.BlockSpec((1, OUT), lambda i: (0, 0)),
        ],
        out_specs=[
            pl.BlockSpec((_NB, OUT), lambda i: (i, 0)),
            pl.BlockSpec((_NB, 8), lambda i: (i, 0)),
            pl.BlockSpec((_NB, 8), lambda i: (i, 0)),
        ],
        out_shape=[_f32((N, OUT)), _f32((N, 8)), _f32((N, 8))],
    )(HH, NUM1, DEN1, b1.reshape(1, -1), W2, ats2.reshape(1, -1), atd2.reshape(1, -1))


def _tc3_body(hh2_ref, num_ref, den_ref, b2_ref, wa_ref, wb_ref, bm1_ref,
              u_ref, v_ref):
    den = den_ref[0] + den_ref[1]
    nk = hh2_ref[...] + num_ref[0] + num_ref[1]
    z = nk / (1.0 + den[:, 0:1] + 1e-16) + b2_ref[...]
    u_ref[...] = z @ wa_ref[...] + bm1_ref[...]
    v_ref[...] = z @ wb_ref[...]


def _tc3(HH2, NUM2, DEN2, b2, Wa, Wb, bm1):
    return pl.pallas_call(
        _tc3_body,
        grid=(N // _NB,),
        in_specs=[
            pl.BlockSpec((_NB, OUT), lambda i: (i, 0)),
            pl.BlockSpec((NC, _NB, OUT), lambda i: (0, i, 0)),
            pl.BlockSpec((NC, _NB, 8), lambda i: (0, i, 0)),
            pl.BlockSpec((1, OUT), lambda i: (0, 0)),
            pl.BlockSpec((OUT, OUT), lambda i: (0, 0)),
            pl.BlockSpec((OUT, OUT), lambda i: (0, 0)),
            pl.BlockSpec((1, OUT), lambda i: (0, 0)),
        ],
        out_specs=[
            pl.BlockSpec((_NB, OUT), lambda i: (i, 0)),
            pl.BlockSpec((_NB, OUT), lambda i: (i, 0)),
        ],
        out_shape=[_f32((N, OUT)), _f32((N, OUT))],
    )(HH2, NUM2, DEN2, b2.reshape(1, -1), Wa, Wb, bm1.reshape(1, -1))


_EB = 2000


def _tc4_body(m_ref, w2_ref, b2_ref, w3_ref, b3_ref, o_ref):
    m2 = jnp.maximum(m_ref[...] @ w2_ref[...] + b2_ref[...], 0.0)
    t = m2 @ w3_ref[...] + b3_ref[...]
    o_ref[...] = jax.nn.sigmoid(t)


def _tc4(M, Wm2, bm2, Wm3, bm3):
    w3 = jnp.pad(Wm3, ((0, 0), (0, 7)))
    b3 = jnp.pad(bm3.reshape(1, 1), ((0, 0), (0, 7)))
    out = pl.pallas_call(
        _tc4_body,
        grid=(E // _EB,),
        in_specs=[
            pl.BlockSpec((_EB, OUT), lambda i: (i, 0)),
            pl.BlockSpec((OUT, OUT // 2), lambda i: (0, 0)),
            pl.BlockSpec((1, OUT // 2), lambda i: (0, 0)),
            pl.BlockSpec((OUT // 2, 8), lambda i: (0, 0)),
            pl.BlockSpec((1, 8), lambda i: (0, 0)),
        ],
        out_specs=pl.BlockSpec((_EB, 8), lambda i: (i, 0)),
        out_shape=_f32((E, 8)),
    )(M, Wm2, bm2.reshape(1, -1), w3, b3)
    return out[:, :1]


def _pad_idx(a):
    return jnp.concatenate([a, jnp.zeros((EPAD - E,), a.dtype)])


def kernel(x, edge_index, edge_type, edge_pairs, rel_emb, W1, att_src1, att_dst1,
           b1, W2, att_src2, att_dst2, b2, Wm1, bm1, Wm2, bm2, Wm3, bm3):
    src = edge_index[0].astype(jnp.int32)
    dst = edge_index[1].astype(jnp.int32)
    et = edge_type.astype(jnp.int32)
    srcp, dstp, etp = _pad_idx(src), _pad_idx(dst), _pad_idx(et)

    cnt2 = _sc_counts(srcp, etp).reshape(NC, N, R)
    HH, AS1, ADC1 = _tc1(x, cnt2, rel_emb, W1, att_src1, att_dst1)
    NUM1, DEN1 = _sc_gat_edges(srcp, dstp, AS1.reshape(-1), ADC1.reshape(-1),
                               HH.reshape(HEADS * N, HID), HEADS)
    HH2, AS2, ADC2 = _tc2(HH, NUM1, DEN1, b1, W2, att_src2, att_dst2)
    NUM2, DEN2 = _sc_gat_edges(srcp, dstp, AS2.reshape(-1), ADC2.reshape(-1),
                               HH2, 1)
    U, V = _tc3(HH2, NUM2.reshape(NC, N, OUT), DEN2, b2, Wm1[:OUT], Wm1[OUT:], bm1)
    ep0 = _pad_idx(edge_pairs[0].astype(jnp.int32))
    ep1 = _pad_idx(edge_pairs[1].astype(jnp.int32))
    M = _sc_pairs(ep0, ep1, U, V)
    return _tc4(M[:E], Wm2, bm2, Wm3, bm3)
